# Initial kernel scaffold; baseline (speedup 1.0000x reference)
import jax
import jax.numpy as jnp
from jax.experimental import pallas as pl


def kernel(x, edge_index, t, idx, W1, b1, g1, bn1, W2, b2, g2, bn2, W3, b3, g3, bn3, W4, b4, g4, bn4, fcW, fcb):
    raise NotImplementedError("write your pallas kernel here")


# speedup vs baseline: 9.5426x; 9.5426x over previous
import jax
import jax.numpy as jnp
from jax import lax
from jax.experimental import pallas as pl
from jax.experimental.pallas import tpu as pltpu
from jax.experimental.pallas import tpu_sc as plsc

N = 10000
NPAD = 10240
E = 160000
CHUNK = 128
NCH = E // CHUNK
NC = 2
NS = 16
ROWS_PER_TILE = NPAD // NS
GPAD = 1024

_MESH = plsc.VectorSubcoreMesh(core_axis_name="c", subcore_axis_name="s")


def _chunk_range(w, nw):
    q, r = NCH // nw, NCH % nw
    base = w * q + jnp.minimum(w, r)
    cnt = q + jnp.where(w < r, 1, 0)
    return base, cnt


def _make_sc_agg(feat_split, with_gather):
    out_type = [jax.ShapeDtypeStruct((NC * NPAD, 128), jnp.float32)]
    scratch = [
        pltpu.VMEM((1, CHUNK), jnp.int32),
        pltpu.VMEM((1, CHUNK), jnp.int32),
        pltpu.VMEM((CHUNK, 128), jnp.float32),
        pltpu.VMEM_SHARED((NPAD, 128), jnp.float32),
    ]
    if with_gather:
        out_type.append(jax.ShapeDtypeStruct((GPAD, 128), jnp.float32))
        scratch += [
            pltpu.VMEM((1, GPAD // (NC * NS)), jnp.int32),
            pltpu.VMEM((GPAD // (NC * NS), 128), jnp.float32),
        ]

    def body(src_h, dst_h, table, zrs, *refs):
        if with_gather:
            (idxh, g2table, out, gout,
             gidx, didx, rows, acc, gi, grows) = refs
        else:
            out, gidx, didx, rows, acc = refs
        c = lax.axis_index("c")
        s = lax.axis_index("s")

        pltpu.sync_copy(zrs, acc.at[pl.ds(s * ROWS_PER_TILE, ROWS_PER_TILE)])
        plsc.subcore_barrier()

        if feat_split:
            base, cnt = _chunk_range(s, NS)
            goff = c * E
        else:
            base, cnt = _chunk_range(s * NC + c, NC * NS)
            goff = 0

        def step(j, _):
            eb = (base + j) * CHUNK
            pltpu.sync_copy(src_h.at[pl.ds(goff + eb, CHUNK)], gidx.at[0])
            pltpu.sync_copy(dst_h.at[pl.ds(eb, CHUNK)], didx.at[0])
            pltpu.sync_copy(table.at[gidx.at[0]], rows)
            pltpu.sync_copy(rows, acc.at[didx.at[0]], add=True)
            return 0

        lax.fori_loop(0, cnt, step, 0)

        if with_gather:
            gper = GPAD // (NC * NS)
            w = s * NC + c
            pltpu.sync_copy(idxh.at[pl.ds(w * gper, gper)], gi.at[0])
            pltpu.sync_copy(g2table.at[gi.at[0]], grows)
            pltpu.sync_copy(grows, gout.at[pl.ds(w * gper, gper)])

        plsc.subcore_barrier()
        r0 = s * ROWS_PER_TILE
        pltpu.sync_copy(acc.at[pl.ds(r0, ROWS_PER_TILE)],
                        out.at[pl.ds(c * NPAD + r0, ROWS_PER_TILE)])

    return pl.kernel(body, out_type=tuple(out_type) if with_gather else out_type[0],
                     mesh=_MESH, scratch_types=scratch)


def _make_sc_deg():
    scratch = [
        pltpu.VMEM((1, CHUNK), jnp.int32),
        pltpu.VMEM((CHUNK, 128), jnp.float32),
        pltpu.VMEM_SHARED((NPAD, 128), jnp.float32),
    ]

    def body(dst_h, ones, zrs, out, didx, rows, acc):
        c = lax.axis_index("c")
        s = lax.axis_index("s")
        pltpu.sync_copy(zrs, acc.at[pl.ds(s * ROWS_PER_TILE, ROWS_PER_TILE)])
        pltpu.sync_copy(ones, rows)
        plsc.subcore_barrier()

        base, cnt = _chunk_range(s * NC + c, NC * NS)

        def step(j, _):
            eb = (base + j) * CHUNK
            pltpu.sync_copy(dst_h.at[pl.ds(eb, CHUNK)], didx.at[0])
            pltpu.sync_copy(rows, acc.at[didx.at[0]], add=True)
            return 0

        lax.fori_loop(0, cnt, step, 0)
        plsc.subcore_barrier()
        r0 = s * ROWS_PER_TILE
        pltpu.sync_copy(acc.at[pl.ds(r0, ROWS_PER_TILE)],
                        out.at[pl.ds(c * NPAD + r0, ROWS_PER_TILE)])

    return pl.kernel(body, out_type=jax.ShapeDtypeStruct((NC * NPAD, 128), jnp.float32),
                     mesh=_MESH, scratch_types=scratch)


BM = 1000
NB = N // BM


def _dinv_col(dinvblk):
    return dinvblk[:, :1]


def _elu(x):
    return jnp.where(x > 0, x, jnp.exp(jnp.minimum(x, 0.0)) - 1.0)


def _ln(o, g, b, eps=1e-5):
    mu = jnp.mean(o, axis=-1, keepdims=True)
    var = jnp.mean((o - mu) * (o - mu), axis=-1, keepdims=True)
    return (o - mu) * lax.rsqrt(var + eps) * g + b


def _split2(p):
    h = p.shape[-1] // 2
    return jnp.stack([p[:, :h], p[:, h:]])


def _tc1_body(x_ref, w_ref, d_ref, o_ref, dv_ref):
    deg = d_ref[0, :, 0] + d_ref[1, :, 0] + 1.0
    dinv = lax.rsqrt(deg)[:, None]
    hw = jnp.dot(x_ref[...], w_ref[...], preferred_element_type=jnp.float32)
    o_ref[0] = hw * dinv
    dv_ref[...] = jnp.broadcast_to(dinv, dv_ref.shape)


def _tc2_body(a, p, d, w2, b1, g1, bn1, o_ref):
    dinv = _dinv_col(d[...])
    aa, pp = a[...], p[...]
    u = jnp.concatenate([aa[0] + pp[0], aa[1] + pp[1]], axis=-1) * dinv
    h = _ln(_elu(u + b1[...]), g1[...], bn1[...])
    o_ref[...] = jnp.dot(h, w2[...], preferred_element_type=jnp.float32) * dinv


def _tc3_body(a, p, d, b2, g2, bn2, h2_ref, p2p_ref):
    dinv = _dinv_col(d[...])
    aa = a[...]
    u = (aa[0] + aa[1] + p[...]) * dinv
    h = _ln(_elu(u + b2[...]), g2[...], bn2[...])
    h2_ref[...] = h
    p2p_ref[...] = h * dinv


def _tc4_body(a, p, d, w3, b3, g3, bn3, p3_ref):
    dinv = _dinv_col(d[...])
    aa = a[...]
    q = (aa[0] + aa[1] + p[...]) * dinv
    o = jnp.dot(q, w3[...], preferred_element_type=jnp.float32) + b3[...]
    h = _ln(_elu(o), g3[...], bn3[...])
    p3_ref[...] = _split2(h * dinv)


def _tc6_body(a, p, d, w4, b4, g4, bn4, h4_ref):
    dinv = _dinv_col(d[...])
    aa, pp = a[...], p[...]
    u = jnp.concatenate([aa[0] + pp[0], aa[1] + pp[1]], axis=-1) * dinv
    o = jnp.dot(u, w4[...], preferred_element_type=jnp.float32) + b4[...]
    h4_ref[...] = _ln(_elu(o), g4[...], bn4[...])


def _tc5_body(h_ref, w_ref, b_ref, o_ref):
    o_ref[...] = jnp.dot(h_ref[...], w_ref[...],
                         preferred_element_type=jnp.float32) + b_ref[...]


def _rows_spec(w):
    return pl.BlockSpec((2, BM, w), lambda i: (0, i, 0))


def _row_spec(w):
    return pl.BlockSpec((BM, w), lambda i: (i, 0))


def _full_spec(shape):
    return pl.BlockSpec(shape, lambda i: tuple(0 for _ in shape))


_DINV_SPEC = pl.BlockSpec((BM, 8), lambda i: (i, 0))


def kernel(x, edge_index, t, idx, W1, b1, g1, bn1, W2, b2, g2, bn2,
           W3, b3, g3, bn3, W4, b4, g4, bn4, fcW, fcb):
    f32 = jnp.float32
    src = edge_index[0]
    dst = edge_index[1]
    src2 = jnp.concatenate([src, src + N])
    idxpad = jnp.concatenate([idx, jnp.zeros((GPAD - idx.shape[0],), idx.dtype)])
    z128 = jnp.zeros((ROWS_PER_TILE, 128), f32)
    ones128 = jnp.ones((CHUNK, 128), f32)

    deg2 = _make_sc_deg()(dst, ones128, z128).reshape(2, NPAD, 128)

    agg_feat = _make_sc_agg(True, False)
    agg_edge = _make_sc_agg(False, False)
    agg_edge_g = _make_sc_agg(False, True)

    b1r, g1r, bn1r = b1.reshape(1, -1), g1.reshape(1, -1), bn1.reshape(1, -1)
    b2r, g2r, bn2r = b2.reshape(1, -1), g2.reshape(1, -1), bn2.reshape(1, -1)
    b3r, g3r, bn3r = b3.reshape(1, -1), g3.reshape(1, -1), bn3.reshape(1, -1)
    b4r, g4r, bn4r = b4.reshape(1, -1), g4.reshape(1, -1), bn4.reshape(1, -1)

    p1, dinv8 = pl.pallas_call(
        _tc1_body,
        grid=(NB, 2),
        in_specs=[
            pl.BlockSpec((BM, 256), lambda i, j: (i, 0)),
            pl.BlockSpec((256, 128), lambda i, j: (0, j)),
            pl.BlockSpec((2, BM, 128), lambda i, j: (0, i, 0)),
        ],
        out_specs=[
            pl.BlockSpec((1, BM, 128), lambda i, j: (j, i, 0)),
            pl.BlockSpec((BM, 8), lambda i, j: (i, 0)),
        ],
        out_shape=[
            jax.ShapeDtypeStruct((2, N, 128), f32),
            jax.ShapeDtypeStruct((N, 8), f32),
        ],
    )(x, W1, deg2)

    a1 = agg_feat(src2, dst, p1.reshape(2 * N, 128), z128).reshape(2, NPAD, 128)

    p2 = pl.pallas_call(
        _tc2_body,
        grid=(NB,),
        in_specs=[
            _rows_spec(128), pl.BlockSpec((2, BM, 128), lambda i: (0, i, 0)),
            _DINV_SPEC,
            _full_spec((256, 128)),
            _full_spec((1, 256)), _full_spec((1, 256)), _full_spec((1, 256)),
        ],
        out_specs=_row_spec(128),
        out_shape=jax.ShapeDtypeStruct((N, 128), f32),
    )(a1, p1.reshape(2, N, 128), dinv8, W2, b1r, g1r, bn1r)

    a2 = agg_edge(src, dst, p2, z128).reshape(2, NPAD, 128)

    h2, p2p = pl.pallas_call(
        _tc3_body,
        grid=(NB,),
        in_specs=[
            _rows_spec(128), _row_spec(128), _DINV_SPEC,
            _full_spec((1, 128)), _full_spec((1, 128)), _full_spec((1, 128)),
        ],
        out_specs=[_row_spec(128), _row_spec(128)],
        out_shape=[
            jax.ShapeDtypeStruct((N, 128), f32),
            jax.ShapeDtypeStruct((N, 128), f32),
        ],
    )(a2, p2, dinv8, b2r, g2r, bn2r)

    a2p, h2g = agg_edge_g(src, dst, p2p, z128, idxpad, h2)
    a2p = a2p.reshape(2, NPAD, 128)

    p3 = pl.pallas_call(
        _tc4_body,
        grid=(NB,),
        in_specs=[
            _rows_spec(128), _row_spec(128), _DINV_SPEC,
            _full_spec((128, 256)),
            _full_spec((1, 256)), _full_spec((1, 256)), _full_spec((1, 256)),
        ],
        out_specs=pl.BlockSpec((2, BM, 128), lambda i: (0, i, 0)),
        out_shape=jax.ShapeDtypeStruct((2, N, 128), f32),
    )(a2p, p2p, dinv8, W3, b3r, g3r, bn3r)

    cp = pl.pallas_call(
        _tc5_body,
        grid=(1,),
        in_specs=[
            pl.BlockSpec((GPAD, 128), lambda i: (0, 0)),
            _full_spec((128, 20)),
            _full_spec((1, 20)),
        ],
        out_specs=pl.BlockSpec((GPAD, 20), lambda i: (0, 0)),
        out_shape=jax.ShapeDtypeStruct((GPAD, 20), f32),
    )(h2g, fcW, fcb.reshape(1, -1))[: idx.shape[0]]

    a3 = agg_feat(src2, dst, p3.reshape(2 * N, 128), z128).reshape(2, NPAD, 128)

    h4 = pl.pallas_call(
        _tc6_body,
        grid=(NB,),
        in_specs=[
            _rows_spec(128), pl.BlockSpec((2, BM, 128), lambda i: (0, i, 0)),
            _DINV_SPEC,
            _full_spec((256, 256)),
            _full_spec((1, 256)), _full_spec((1, 256)), _full_spec((1, 256)),
        ],
        out_specs=pl.BlockSpec((BM, 256), lambda i: (i, 0)),
        out_shape=jax.ShapeDtypeStruct((N, 256), f32),
    )(a3, p3.reshape(2, N, 128), dinv8, W4, b4r, g4r, bn4r)

    return (h2, h4, cp)
